# baseline (speedup 1.0000x reference)
import functools

import jax
import jax.numpy as jnp
from jax import lax
from jax.experimental import pallas as pl
from jax.experimental.pallas import tpu as pltpu
from jax.experimental.pallas import tpu_sc as plsc

_N = 100000
_D = 128
_R = 10000
_NBLK = _N // _R

_NW = 32
_WPW = _N * _D // _NW
_CHUNK = 16000
_NCHUNK = _WPW // _CHUNK


def _tc_dots(z_ref, zp_ref, zn_ref, acc_ref):
    i = pl.program_id(0)

    @pl.when(i == 0)
    def _init():
        acc_ref[0] = 0.0
        acc_ref[1] = 0.0

    z = z_ref[...]
    ones_row = jnp.ones((1, _D), dtype=jnp.float32)
    dnums = (((1,), (1,)), ((), ()))
    pdot = jax.lax.dot_general(ones_row, z * zp_ref[...], dnums,
                               preferred_element_type=jnp.float32)
    ndot = jax.lax.dot_general(ones_row, z * zn_ref[...], dnums,
                               preferred_element_type=jnp.float32)
    acc_ref[0] += jnp.sum(jax.nn.log_sigmoid(pdot))
    acc_ref[1] += jnp.sum(jax.nn.log_sigmoid(-ndot))


@functools.partial(
    pl.kernel,
    out_type=jax.ShapeDtypeStruct((_NW, 16), jnp.float32),
    mesh=plsc.VectorSubcoreMesh(core_axis_name="c", subcore_axis_name="s"),
    scratch_types=[
        pltpu.VMEM((_CHUNK,), jnp.float32),
        pltpu.VMEM((_CHUNK,), jnp.float32),
        pltpu.VMEM((16,), jnp.float32),
    ],
)
def _sc_mse(x_hbm, xh_hbm, out_hbm, xv, xhv, accv):
    wid = lax.axis_index("s") * 2 + lax.axis_index("c")
    base = wid * _WPW

    def chunk_body(c, acc):
        off = base + c * _CHUNK
        pltpu.sync_copy(x_hbm.at[pl.ds(off, _CHUNK)], xv)
        pltpu.sync_copy(xh_hbm.at[pl.ds(off, _CHUNK)], xhv)

        def w(k, a):
            d = xv[pl.ds(k * 16, 16)] - xhv[pl.ds(k * 16, 16)]
            return a + d * d

        return lax.fori_loop(0, _CHUNK // 16, w, acc, unroll=8)

    acc = lax.fori_loop(0, _NCHUNK, chunk_body,
                        jnp.zeros((16,), jnp.float32))
    accv[...] = acc
    pltpu.sync_copy(accv, out_hbm.at[wid])


def kernel(out, x_full, xhat_full, lamb):
    tc_sums = pl.pallas_call(
        _tc_dots,
        grid=(_NBLK,),
        in_specs=[
            pl.BlockSpec((_R, _D), lambda i: (i, 0)),
            pl.BlockSpec((_R, _D), lambda i: (i + _NBLK, 0)),
            pl.BlockSpec((_R, _D), lambda i: (i + 2 * _NBLK, 0)),
        ],
        out_specs=pl.BlockSpec(memory_space=pltpu.SMEM),
        out_shape=jax.ShapeDtypeStruct((2,), jnp.float32),
    )(out, out, out)

    mse_parts = _sc_mse(x_full.reshape(-1), xhat_full.reshape(-1))

    lamb = jnp.clip(lamb, 1e-08, 1.0 - 1e-08)
    pos_loss = tc_sums[0] / _N
    neg_loss = tc_sums[1] / _N
    mse = jnp.sum(mse_parts) / (_N * _D)
    return lamb * mse + (1.0 - lamb) * (-pos_loss - neg_loss)


# speedup vs baseline: 1.2710x; 1.2710x over previous
import functools

import jax
import jax.numpy as jnp
from jax import lax
from jax.experimental import pallas as pl
from jax.experimental.pallas import tpu as pltpu
from jax.experimental.pallas import tpu_sc as plsc

_N = 100000
_D = 128
_R = 10000
_NBLK = _N // _R

_NW = 32
_WPW = _N * _D // _NW
_CHUNK = 20000
_NCHUNK = _WPW // _CHUNK


def _tc_dots(z_ref, zp_ref, zn_ref, acc_ref):
    i = pl.program_id(0)

    @pl.when(i == 0)
    def _init():
        acc_ref[0] = 0.0
        acc_ref[1] = 0.0

    z = z_ref[...]
    ones_row = jnp.ones((1, _D), dtype=jnp.float32)
    dnums = (((1,), (1,)), ((), ()))
    pdot = jax.lax.dot_general(ones_row, z * zp_ref[...], dnums,
                               preferred_element_type=jnp.float32)
    ndot = jax.lax.dot_general(ones_row, z * zn_ref[...], dnums,
                               preferred_element_type=jnp.float32)
    acc_ref[0] += jnp.sum(jax.nn.log_sigmoid(pdot))
    acc_ref[1] += jnp.sum(jax.nn.log_sigmoid(-ndot))


@functools.partial(
    pl.kernel,
    out_type=jax.ShapeDtypeStruct((_NW, 16), jnp.float32),
    mesh=plsc.VectorSubcoreMesh(core_axis_name="c", subcore_axis_name="s"),
    scratch_types=[
        pltpu.VMEM((_CHUNK,), jnp.float32),
        pltpu.VMEM((_CHUNK,), jnp.float32),
        pltpu.VMEM((_CHUNK,), jnp.float32),
        pltpu.VMEM((_CHUNK,), jnp.float32),
        pltpu.VMEM((16,), jnp.float32),
        pltpu.SemaphoreType.DMA,
        pltpu.SemaphoreType.DMA,
    ],
)
def _sc_mse(x_hbm, xh_hbm, out_hbm, xv0, xv1, xhv0, xhv1, accv, sem0, sem1):
    wid = lax.axis_index("s") * 2 + lax.axis_index("c")
    base = wid * _WPW
    bufs = ((xv0, xhv0, sem0), (xv1, xhv1, sem1))

    def start(b, c):
        xv, xhv, sem = bufs[b]
        off = base + c * _CHUNK
        pltpu.make_async_copy(x_hbm.at[pl.ds(off, _CHUNK)], xv, sem).start()
        pltpu.make_async_copy(xh_hbm.at[pl.ds(off, _CHUNK)], xhv, sem).start()

    def wait_compute(b, c, acc):
        xv, xhv, sem = bufs[b]
        off = base + c * _CHUNK
        pltpu.make_async_copy(x_hbm.at[pl.ds(off, _CHUNK)], xv, sem).wait()
        pltpu.make_async_copy(xh_hbm.at[pl.ds(off, _CHUNK)], xhv, sem).wait()

        def w(k, a):
            d = xv[pl.ds(k * 16, 16)] - xhv[pl.ds(k * 16, 16)]
            return a + d * d

        return lax.fori_loop(0, _CHUNK // 16, w, acc, unroll=8)

    start(0, 0)
    start(1, 1)

    def pair(k, acc):
        c = 2 * k
        acc = wait_compute(0, c, acc)
        start(0, c + 2)
        acc = wait_compute(1, c + 1, acc)
        start(1, c + 3)
        return acc

    acc = lax.fori_loop(0, _NCHUNK // 2 - 1, pair,
                        jnp.zeros((16,), jnp.float32))
    acc = wait_compute(0, _NCHUNK - 2, acc)
    acc = wait_compute(1, _NCHUNK - 1, acc)
    accv[...] = acc
    pltpu.sync_copy(accv, out_hbm.at[wid])


def kernel(out, x_full, xhat_full, lamb):
    tc_sums = pl.pallas_call(
        _tc_dots,
        grid=(_NBLK,),
        in_specs=[
            pl.BlockSpec((_R, _D), lambda i: (i, 0)),
            pl.BlockSpec((_R, _D), lambda i: (i + _NBLK, 0)),
            pl.BlockSpec((_R, _D), lambda i: (i + 2 * _NBLK, 0)),
        ],
        out_specs=pl.BlockSpec(memory_space=pltpu.SMEM),
        out_shape=jax.ShapeDtypeStruct((2,), jnp.float32),
    )(out, out, out)

    mse_parts = _sc_mse(x_full.reshape(-1), xhat_full.reshape(-1))

    lamb = jnp.clip(lamb, 1e-08, 1.0 - 1e-08)
    pos_loss = tc_sums[0] / _N
    neg_loss = tc_sums[1] / _N
    mse = jnp.sum(mse_parts) / (_N * _D)
    return lamb * mse + (1.0 - lamb) * (-pos_loss - neg_loss)


# speedup vs baseline: 1.2731x; 1.0017x over previous
import functools

import jax
import jax.numpy as jnp
from jax import lax
from jax.experimental import pallas as pl
from jax.experimental.pallas import tpu as pltpu
from jax.experimental.pallas import tpu_sc as plsc

_N = 100000
_D = 128

_NSC = 80000
_R = 10000
_NBLK = _N // _R
_RT = (_N - _NSC) // _NBLK

_NW = 32
_WPW = _NSC * _D // _NW
_CHUNK = 20000
_NCHUNK = _WPW // _CHUNK


def _tc_body(z_ref, zp_ref, zn_ref, x_ref, xh_ref, acc_ref):
    i = pl.program_id(0)

    @pl.when(i == 0)
    def _init():
        acc_ref[0] = 0.0
        acc_ref[1] = 0.0
        acc_ref[2] = 0.0

    z = z_ref[...]
    ones_row = jnp.ones((1, _D), dtype=jnp.float32)
    dnums = (((1,), (1,)), ((), ()))
    pdot = jax.lax.dot_general(ones_row, z * zp_ref[...], dnums,
                               preferred_element_type=jnp.float32)
    ndot = jax.lax.dot_general(ones_row, z * zn_ref[...], dnums,
                               preferred_element_type=jnp.float32)
    acc_ref[0] += jnp.sum(jax.nn.log_sigmoid(pdot))
    acc_ref[1] += jnp.sum(jax.nn.log_sigmoid(-ndot))
    diff = x_ref[...] - xh_ref[...]
    acc_ref[2] += jnp.sum(diff * diff)


@functools.partial(
    pl.kernel,
    out_type=jax.ShapeDtypeStruct((_NW, 16), jnp.float32),
    mesh=plsc.VectorSubcoreMesh(core_axis_name="c", subcore_axis_name="s"),
    scratch_types=[
        pltpu.VMEM((_CHUNK,), jnp.float32),
        pltpu.VMEM((_CHUNK,), jnp.float32),
        pltpu.VMEM((_CHUNK,), jnp.float32),
        pltpu.VMEM((_CHUNK,), jnp.float32),
        pltpu.VMEM((16,), jnp.float32),
        pltpu.SemaphoreType.DMA,
        pltpu.SemaphoreType.DMA,
    ],
)
def _sc_mse(x_hbm, xh_hbm, out_hbm, xv0, xv1, xhv0, xhv1, accv, sem0, sem1):
    wid = lax.axis_index("s") * 2 + lax.axis_index("c")
    base = wid * _WPW
    bufs = ((xv0, xhv0, sem0), (xv1, xhv1, sem1))

    def start(b, c):
        xv, xhv, sem = bufs[b]
        off = base + c * _CHUNK
        pltpu.make_async_copy(x_hbm.at[pl.ds(off, _CHUNK)], xv, sem).start()
        pltpu.make_async_copy(xh_hbm.at[pl.ds(off, _CHUNK)], xhv, sem).start()

    def wait_compute(b, c, acc):
        xv, xhv, sem = bufs[b]
        off = base + c * _CHUNK
        pltpu.make_async_copy(x_hbm.at[pl.ds(off, _CHUNK)], xv, sem).wait()
        pltpu.make_async_copy(xh_hbm.at[pl.ds(off, _CHUNK)], xhv, sem).wait()

        def w(k, a):
            d = xv[pl.ds(k * 16, 16)] - xhv[pl.ds(k * 16, 16)]
            return a + d * d

        return lax.fori_loop(0, _CHUNK // 16, w, acc, unroll=8)

    start(0, 0)
    start(1, 1)

    def pair(k, acc):
        c = 2 * k
        acc = wait_compute(0, c, acc)
        start(0, c + 2)
        acc = wait_compute(1, c + 1, acc)
        start(1, c + 3)
        return acc

    acc = lax.fori_loop(0, _NCHUNK // 2 - 1, pair,
                        jnp.zeros((16,), jnp.float32))
    acc = wait_compute(0, _NCHUNK - 2, acc)
    acc = wait_compute(1, _NCHUNK - 1, acc)
    accv[...] = acc
    pltpu.sync_copy(accv, out_hbm.at[wid])


def kernel(out, x_full, xhat_full, lamb):
    tail_spec = pl.BlockSpec((_RT, _D), lambda i: (i + _NSC // _RT, 0))
    tc_sums = pl.pallas_call(
        _tc_body,
        grid=(_NBLK,),
        in_specs=[
            pl.BlockSpec((_R, _D), lambda i: (i, 0)),
            pl.BlockSpec((_R, _D), lambda i: (i + _NBLK, 0)),
            pl.BlockSpec((_R, _D), lambda i: (i + 2 * _NBLK, 0)),
            tail_spec,
            tail_spec,
        ],
        out_specs=pl.BlockSpec(memory_space=pltpu.SMEM),
        out_shape=jax.ShapeDtypeStruct((3,), jnp.float32),
    )(out, out, out, x_full, xhat_full)

    mse_parts = _sc_mse(x_full.reshape(-1), xhat_full.reshape(-1))

    lamb = jnp.clip(lamb, 1e-08, 1.0 - 1e-08)
    pos_loss = tc_sums[0] / _N
    neg_loss = tc_sums[1] / _N
    mse = (jnp.sum(mse_parts) + tc_sums[2]) / (_N * _D)
    return lamb * mse + (1.0 - lamb) * (-pos_loss - neg_loss)


# speedup vs baseline: 1.2732x; 1.0001x over previous
import functools

import jax
import jax.numpy as jnp
from jax import lax
from jax.experimental import pallas as pl
from jax.experimental.pallas import tpu as pltpu
from jax.experimental.pallas import tpu_sc as plsc

_N = 100000
_D = 128

_NSC = 80000
_R = 10000
_NBLK = _N // _R
_RT = (_N - _NSC) // _NBLK

_NW = 32
_WPW = _NSC * _D // _NW
_CHUNK = 20000
_NCHUNK = _WPW // _CHUNK


def _tc_body(z_ref, zp_ref, zn_ref, x_ref, xh_ref, acc_ref):
    i = pl.program_id(0)

    @pl.when(i == 0)
    def _init():
        acc_ref[0] = 0.0
        acc_ref[1] = 0.0
        acc_ref[2] = 0.0

    z = z_ref[...]
    ones_row = jnp.ones((1, _D), dtype=jnp.float32)
    dnums = (((1,), (1,)), ((), ()))
    pdot = jax.lax.dot_general(ones_row, z * zp_ref[...], dnums,
                               preferred_element_type=jnp.float32)
    ndot = jax.lax.dot_general(ones_row, z * zn_ref[...], dnums,
                               preferred_element_type=jnp.float32)
    acc_ref[0] += jnp.sum(jax.nn.log_sigmoid(pdot))
    acc_ref[1] += jnp.sum(jax.nn.log_sigmoid(-ndot))
    diff = x_ref[...] - xh_ref[...]
    acc_ref[2] += jnp.sum(diff * diff)


@functools.partial(
    pl.kernel,
    out_type=jax.ShapeDtypeStruct((_NW, 16), jnp.float32),
    mesh=plsc.VectorSubcoreMesh(core_axis_name="c", subcore_axis_name="s"),
    cost_estimate=pl.CostEstimate(
        flops=2 * _NSC * _D,
        bytes_accessed=2 * _NSC * _D * 4,
        transcendentals=0,
    ),
    scratch_types=[
        pltpu.VMEM((_CHUNK,), jnp.float32),
        pltpu.VMEM((_CHUNK,), jnp.float32),
        pltpu.VMEM((_CHUNK,), jnp.float32),
        pltpu.VMEM((_CHUNK,), jnp.float32),
        pltpu.VMEM((16,), jnp.float32),
        pltpu.SemaphoreType.DMA,
        pltpu.SemaphoreType.DMA,
    ],
)
def _sc_mse(x_hbm, xh_hbm, out_hbm, xv0, xv1, xhv0, xhv1, accv, sem0, sem1):
    wid = lax.axis_index("s") * 2 + lax.axis_index("c")
    base = wid * _WPW
    bufs = ((xv0, xhv0, sem0), (xv1, xhv1, sem1))

    def start(b, c):
        xv, xhv, sem = bufs[b]
        off = base + c * _CHUNK
        pltpu.make_async_copy(x_hbm.at[pl.ds(off, _CHUNK)], xv, sem).start()
        pltpu.make_async_copy(xh_hbm.at[pl.ds(off, _CHUNK)], xhv, sem).start()

    def wait_compute(b, c, acc):
        xv, xhv, sem = bufs[b]
        off = base + c * _CHUNK
        pltpu.make_async_copy(x_hbm.at[pl.ds(off, _CHUNK)], xv, sem).wait()
        pltpu.make_async_copy(xh_hbm.at[pl.ds(off, _CHUNK)], xhv, sem).wait()

        def w(k, a):
            d = xv[pl.ds(k * 16, 16)] - xhv[pl.ds(k * 16, 16)]
            return a + d * d

        return lax.fori_loop(0, _CHUNK // 16, w, acc, unroll=8)

    start(0, 0)
    start(1, 1)

    def pair(k, acc):
        c = 2 * k
        acc = wait_compute(0, c, acc)
        start(0, c + 2)
        acc = wait_compute(1, c + 1, acc)
        start(1, c + 3)
        return acc

    acc = lax.fori_loop(0, _NCHUNK // 2 - 1, pair,
                        jnp.zeros((16,), jnp.float32))
    acc = wait_compute(0, _NCHUNK - 2, acc)
    acc = wait_compute(1, _NCHUNK - 1, acc)
    accv[...] = acc
    pltpu.sync_copy(accv, out_hbm.at[wid])


def kernel(out, x_full, xhat_full, lamb):
    mse_parts = _sc_mse(x_full.reshape(-1), xhat_full.reshape(-1))

    tail_spec = pl.BlockSpec((_RT, _D), lambda i: (i + _NSC // _RT, 0))
    tc_sums = pl.pallas_call(
        _tc_body,
        grid=(_NBLK,),
        in_specs=[
            pl.BlockSpec((_R, _D), lambda i: (i, 0)),
            pl.BlockSpec((_R, _D), lambda i: (i + _NBLK, 0)),
            pl.BlockSpec((_R, _D), lambda i: (i + 2 * _NBLK, 0)),
            tail_spec,
            tail_spec,
        ],
        out_specs=pl.BlockSpec(memory_space=pltpu.SMEM),
        out_shape=jax.ShapeDtypeStruct((3,), jnp.float32),
        cost_estimate=pl.CostEstimate(
            flops=8 * _N * _D,
            bytes_accessed=(3 * _N + 2 * (_N - _NSC)) * _D * 4,
            transcendentals=2 * _N,
        ),
    )(out, out, out, x_full, xhat_full)

    lamb = jnp.clip(lamb, 1e-08, 1.0 - 1e-08)
    pos_loss = tc_sums[0] / _N
    neg_loss = tc_sums[1] / _N
    mse = (jnp.sum(mse_parts) + tc_sums[2]) / (_N * _D)
    return lamb * mse + (1.0 - lamb) * (-pos_loss - neg_loss)


# speedup vs baseline: 1.6009x; 1.2573x over previous
import jax
import jax.numpy as jnp
from jax.experimental import pallas as pl
from jax.experimental.pallas import tpu as pltpu

_N = 100000
_D = 128
_R = 5000
_NBLK = _N // _R


def _body(z_ref, zp_ref, zn_ref, x_ref, xh_ref, acc_ref):
    i = pl.program_id(0)

    @pl.when(i == 0)
    def _init():
        acc_ref[0] = 0.0
        acc_ref[1] = 0.0
        acc_ref[2] = 0.0

    z = z_ref[...]
    ones_row = jnp.ones((1, _D), dtype=jnp.float32)
    dnums = (((1,), (1,)), ((), ()))
    pdot = jax.lax.dot_general(ones_row, z * zp_ref[...], dnums,
                               preferred_element_type=jnp.float32)
    ndot = jax.lax.dot_general(ones_row, z * zn_ref[...], dnums,
                               preferred_element_type=jnp.float32)
    pos_part = jnp.sum(jax.nn.log_sigmoid(pdot))
    neg_part = jnp.sum(jax.nn.log_sigmoid(-ndot))
    diff = x_ref[...] - xh_ref[...]
    mse_part = jnp.sum(diff * diff)
    acc_ref[0] += pos_part
    acc_ref[1] += neg_part
    acc_ref[2] += mse_part


def kernel(out, x_full, xhat_full, lamb):
    row_spec = pl.BlockSpec((_R, _D), lambda i: (i, 0))
    sums = pl.pallas_call(
        _body,
        grid=(_NBLK,),
        in_specs=[
            pl.BlockSpec((_R, _D), lambda i: (i, 0)),
            pl.BlockSpec((_R, _D), lambda i: (i + _NBLK, 0)),
            pl.BlockSpec((_R, _D), lambda i: (i + 2 * _NBLK, 0)),
            row_spec,
            row_spec,
        ],
        out_specs=pl.BlockSpec(memory_space=pltpu.SMEM),
        out_shape=jax.ShapeDtypeStruct((3,), jnp.float32),
    )(out, out, out, x_full, xhat_full)

    lamb = jnp.clip(lamb, 1e-08, 1.0 - 1e-08)
    pos_loss = sums[0] / _N
    neg_loss = sums[1] / _N
    mse = sums[2] / (_N * _D)
    return lamb * mse + (1.0 - lamb) * (-pos_loss - neg_loss)
